# Initial kernel scaffold; baseline (speedup 1.0000x reference)
import jax
import jax.numpy as jnp
from jax.experimental import pallas as pl


def kernel(queries, keys, values, attn_mask):
    raise NotImplementedError("write your pallas kernel here")


# speedup vs baseline: 4.2617x; 4.2617x over previous
import functools

import jax
import jax.numpy as jnp
import numpy as np
from jax import lax
from jax.experimental import pallas as pl

_B, _H, _D, _L = 2, 12, 64, 2048
_U_PART = 25
_U_TOP = 40
_SEL_PAD = 128
_C_SCALE = 1.0 / (64.0 * 64.0 * 2048.0)
_RB = 256
_CB = 256

_HIGH = lax.Precision.HIGHEST


def _threefry2x32(k1, k2, x0, x1):
    rot_a = (13, 15, 26, 6)
    rot_b = (17, 29, 16, 24)
    u32 = np.uint32
    ks = [u32(k1), u32(k2), u32(k1) ^ u32(k2) ^ u32(0x1BD11BDA)]
    x0 = x0 + ks[0]
    x1 = x1 + ks[1]

    def four_rounds(x0, x1, rots):
        for r in rots:
            x0 = x0 + x1
            x1 = (x1 << u32(r)) | (x1 >> u32(32 - r))
            x1 = x0 ^ x1
        return x0, x1

    inject = [(1, 2), (2, 0), (0, 1), (1, 2), (2, 0)]
    for i, (a, b) in enumerate(inject):
        x0, x1 = four_rounds(x0, x1, rot_a if i % 2 == 0 else rot_b)
        x0 = x0 + ks[a]
        x1 = x1 + ks[b] + u32(i + 1)
    return x0, x1


def _sample_count_matrix() -> np.ndarray:
    u32 = np.uint32
    b1, b2 = _threefry2x32(u32(0), u32(12345),
                           np.zeros(2, np.uint32), np.arange(2, dtype=np.uint32))
    n = _L * _U_PART
    bb1, bb2 = _threefry2x32(b1[1], b2[1],
                             np.zeros(n, np.uint32), np.arange(n, dtype=np.uint32))
    idx = ((bb1 ^ bb2) % u32(_L)).astype(np.int64).reshape(_L, _U_PART)
    cnt = np.zeros((_L, _L), np.float32)
    np.add.at(cnt, (np.arange(_L)[:, None], idx), 1.0)
    return cnt.astype(jnp.bfloat16)


_CNT = _sample_count_matrix()


def _head_body(q_ref, k_ref, v_ref, cnt_ref, o_ref):
    q = q_ref[0, 0]
    k = k_ref[0, 0]
    v = v_ref[0, 0]

    nb = _L // _RB
    m_cols = []
    for rb in range(nb):
        qb = q[:, rb * _RB:(rb + 1) * _RB]
        s = lax.dot_general(qb, k, (((0,), (0,)), ((), ())),
                            preferred_element_type=jnp.float32,
                            precision=_HIGH)
        cb = cnt_ref[rb * _RB:(rb + 1) * _RB, :].astype(jnp.float32)
        mx = jnp.max(jnp.where(cb > 0.0, s, -jnp.inf), axis=1, keepdims=True)
        sm = jnp.sum(s * cb, axis=1, keepdims=True)
        m_cols.append(mx - sm * (1.0 / _L))
    m0 = jnp.concatenate(m_cols, axis=1)

    lmap = (lax.broadcasted_iota(jnp.int32, (_RB, nb), 1) * _RB
            + lax.broadcasted_iota(jnp.int32, (_RB, nb), 0))
    lane_i = lax.broadcasted_iota(jnp.int32, (1, _SEL_PAD), 1)

    def step(i, carry):
        mv, sel = carry
        m = jnp.max(jnp.max(mv, axis=1, keepdims=True), axis=0, keepdims=True)
        flat = jnp.min(jnp.min(jnp.where(mv == m, lmap, jnp.int32(1 << 30)),
                               axis=1, keepdims=True), axis=0, keepdims=True)
        sel = jnp.where(lane_i == i, flat, sel)
        mv = jnp.where(lmap == flat, -jnp.inf, mv)
        return mv, sel

    sel0 = jnp.full((1, _SEL_PAD), -1, jnp.int32)
    _, sel = lax.fori_loop(0, _U_TOP, step, (m0, sel0))

    j_sub = lax.broadcasted_iota(jnp.int32, (_L, _SEL_PAD), 0)
    onehot = (j_sub == sel).astype(jnp.float32)

    q_sel = lax.dot_general(q, onehot, (((1,), (0,)), ((), ())),
                            preferred_element_type=jnp.float32,
                            precision=_HIGH)
    scores = lax.dot_general(q_sel, k, (((0,), (0,)), ((), ())),
                             preferred_element_type=jnp.float32,
                             precision=_HIGH)

    jvalf = lax.broadcasted_iota(jnp.int32, (_L, 1), 0).astype(jnp.float32)
    sel_t = lax.dot_general(onehot, jvalf, (((0,), (0,)), ((), ())),
                            preferred_element_type=jnp.float32,
                            precision=_HIGH)

    lane_j = lax.broadcasted_iota(jnp.int32, (_SEL_PAD, _L), 1).astype(jnp.float32)
    logits = jnp.where(lane_j > sel_t, -jnp.inf, scores * _C_SCALE)
    lmax = jnp.max(logits, axis=1, keepdims=True)
    e = jnp.exp(logits - lmax)
    attn = e / jnp.sum(e, axis=1, keepdims=True)
    upd = lax.dot_general(attn, v, (((1,), (1,)), ((), ())),
                          preferred_element_type=jnp.float32,
                          precision=_HIGH)

    scat = lax.dot_general(upd, onehot, (((0,), (1,)), ((), ())),
                           preferred_element_type=jnp.float32,
                           precision=_HIGH)
    colmask = lax.dot_general(jnp.ones((1, _SEL_PAD), jnp.float32), onehot,
                              (((1,), (1,)), ((), ())),
                              preferred_element_type=jnp.float32,
                              precision=_HIGH)

    tri = (lax.broadcasted_iota(jnp.int32, (_CB, _CB), 0)
           <= lax.broadcasted_iota(jnp.int32, (_CB, _CB), 1)).astype(jnp.float32)
    carry = jnp.zeros((_D, 1), jnp.float32)
    for cb in range(_L // _CB):
        sl = slice(cb * _CB, (cb + 1) * _CB)
        vb = v[:, sl]
        ctx = lax.dot_general(vb, tri, (((1,), (0,)), ((), ())),
                              preferred_element_type=jnp.float32,
                              precision=_HIGH) + carry
        carry = carry + jnp.sum(vb, axis=1, keepdims=True)
        out = jnp.where(colmask[:, sl] > 0.5, scat[:, sl], ctx)
        o_ref[0, 0, :, sl] = out * (1.0 / 64.0)


@jax.jit
def kernel(queries, keys, values, attn_mask):
    del attn_mask
    head_spec = pl.BlockSpec((1, 1, _D, _L), lambda b, h: (b, h, 0, 0))
    cnt_spec = pl.BlockSpec((_L, _L), lambda b, h: (0, 0))
    return pl.pallas_call(
        _head_body,
        grid=(_B, _H),
        in_specs=[head_spec, head_spec, head_spec, cnt_spec],
        out_specs=head_spec,
        out_shape=jax.ShapeDtypeStruct((_B, _H, _D, _L), jnp.float32),
    )(queries, keys, values, _CNT)
